# baseline (speedup 1.0000x reference)
import functools

import jax
import jax.numpy as jnp
from jax import lax
from jax.experimental import pallas as pl
from jax.experimental.pallas import tpu as pltpu
from jax.experimental.pallas import tpu_sc as plsc

_LAMBDA_C = 1.0
_CH = 256


@functools.lru_cache(maxsize=None)
def _build_sc_partials(batch: int, dim: int, num_classes: int):
    info = plsc.get_sparse_core_info()
    nc, ns, lanes = info.num_cores, info.num_subcores, info.num_lanes
    nw = nc * ns
    assert dim == 64 and lanes == 16
    main = (num_classes // 128) * 128
    tail = num_classes - main
    nch = main // _CH
    nt = (nch + nw - 1) // nw

    mesh = plsc.VectorSubcoreMesh(core_axis_name="c", subcore_axis_name="s")

    @functools.partial(
        pl.kernel,
        mesh=mesh,
        out_type=jax.ShapeDtypeStruct((nw * lanes,), jnp.float32),
        compiler_params=pltpu.CompilerParams(needs_layout_passes=False),
        scratch_types=[
            pltpu.VMEM((batch,), jnp.int32),
            pltpu.VMEM((batch,), jnp.int32),
            pltpu.VMEM((batch,), jnp.int32),
            pltpu.VMEM((dim, _CH), jnp.float32),
            pltpu.VMEM((dim, _CH), jnp.float32),
            pltpu.VMEM((tail, dim), jnp.float32),
            pltpu.VMEM((lanes, dim), jnp.float32),
            pltpu.VMEM((lanes,), jnp.float32),
            pltpu.HBM((dim, _CH), jnp.float32),
            pltpu.HBM((lanes, dim), jnp.float32),
            pltpu.SemaphoreType.DMA,
            pltpu.SemaphoreType.DMA,
            pltpu.SemaphoreType.DMA,
        ],
    )
    def sc_partials(y_hbm, hidden_hbm, centers_t_hbm, tail_hbm, out_hbm,
                    y_v, list_c, list_b, ck0, ck1, tail_v, hrow_v, acc_v,
                    dummy_ck, dummy_hr, sem0, sem1, hsem):
        wid = lax.axis_index("s") * nc + lax.axis_index("c")
        pltpu.sync_copy(y_hbm, y_v)
        pltpu.sync_copy(tail_hbm, tail_v)
        lane_iota = lax.iota(jnp.int32, lanes)

        wid_vec = lax.broadcast(wid.astype(jnp.int32), (lanes,))
        is_last_w = wid_vec == (nw - 1)
        ones_i = jnp.ones((lanes,), jnp.int32)
        zeros_i = jnp.zeros((lanes,), jnp.int32)

        def bucket(i, cnt):
            yv = y_v[pl.ds(i * lanes, lanes)]
            cid = yv // _CH
            m = jnp.logical_or(
                jnp.logical_and(cid % nw == wid_vec, yv < main),
                jnp.logical_and(yv >= main, is_last_w))
            mi = jnp.where(m, ones_i, zeros_i)
            pos = lax.broadcast(cnt, (lanes,)) + plsc.cumsum(mi) - 1
            plsc.store_scatter(list_c, [pos], yv, mask=m)
            plsc.store_scatter(list_b, [pos], i * lanes + lane_iota, mask=m)
            return cnt + plsc.all_reduce_population_count(m)[0]

        cnt = lax.fori_loop(0, batch // lanes, bucket, jnp.int32(0))
        ng = (cnt + lanes - 1) // lanes
        cnt_vec = lax.broadcast(cnt, (lanes,))

        def process_group(g, acc, lo, width, table_v, col_of):
            c_vec = list_c[pl.ds(g * lanes, lanes)]
            b_vec = list_b[pl.ds(g * lanes, lanes)]
            m = jnp.logical_and(
                jnp.logical_and(c_vec >= lo, c_vec < lo + width),
                (g * lanes + lane_iota) < cnt_vec)
            hits = plsc.all_reduce_population_count(m)[0]

            def do(acc):
                wf = jnp.where(m, jnp.ones((lanes,), jnp.float32),
                               jnp.zeros((lanes,), jnp.float32))
                b_safe = jnp.clip(b_vec, 0, batch - 1)
                col = jnp.clip(c_vec - lo, 0, width - 1)
                for t in range(lanes):
                    pltpu.async_copy(hidden_hbm.at[b_safe[t]],
                                     hrow_v.at[t], hsem)
                pltpu.make_async_copy(dummy_hr, hrow_v, hsem).wait()
                for t in range(lanes):
                    w_t = wf[t]
                    col_t = col[t]
                    for k in range(dim // lanes):
                        cv = plsc.load_gather(
                            table_v, col_of(col_t, k * lanes + lane_iota))
                        hv = hrow_v[t, pl.ds(k * lanes, lanes)]
                        d = (hv - cv) * w_t
                        acc = acc + d * d
                return acc

            return lax.cond(hits > 0, do, lambda a: a, acc)

        def fire_chunk(slot, buf, sem):
            c_idx = wid + slot * nw

            @pl.when(c_idx < nch)
            def _():
                c0 = pl.multiple_of(c_idx * _CH, _CH)
                for i in range(dim // 8):
                    pltpu.async_copy(
                        centers_t_hbm.at[pl.ds(8 * i, 8), pl.ds(c0, _CH)],
                        buf.at[pl.ds(8 * i, 8), :], sem)

        fire_chunk(0, ck0, sem0)
        zero = jnp.zeros((lanes,), jnp.float32)

        def chunk_body(t, buf, sem, nbuf, nsem, acc):
            c_idx = wid + t * nw

            def do(acc):
                pltpu.make_async_copy(dummy_ck, buf, sem).wait()
                fire_chunk(t + 1, nbuf, nsem)
                lo = c_idx * _CH

                def grp(g, a):
                    return process_group(
                        g, a, lo, _CH, buf,
                        lambda col_t, dvec: [dvec, lax.broadcast(
                            col_t, (lanes,))])

                return lax.fori_loop(0, ng, grp, acc)

            return lax.cond(c_idx < nch, do, lambda a: a, acc)

        def pair_body(t2, acc):
            acc = chunk_body(2 * t2, ck0, sem0, ck1, sem1, acc)
            acc = chunk_body(2 * t2 + 1, ck1, sem1, ck0, sem0, acc)
            return acc

        acc = lax.fori_loop(0, (nt + 1) // 2, pair_body, zero)

        def tail_grp(g, a):
            return process_group(
                g, a, main, tail, tail_v,
                lambda col_t, dvec: [lax.broadcast(col_t, (lanes,)), dvec])

        acc = lax.fori_loop(0, ng, tail_grp, acc)

        acc_v[...] = acc
        pltpu.sync_copy(acc_v, out_hbm.at[pl.ds(wid * lanes, lanes)])

    return sc_partials, nw, lanes, main


def _finish_body(scale_sq, p_ref, o_ref):
    s = jnp.sum(p_ref[...])
    o_ref[...] = jnp.broadcast_to(jnp.sqrt(s * scale_sq), (1, 1))


def kernel(y, hidden, centers):
    batch, dim = hidden.shape
    num_classes = centers.shape[0]
    sc_partials, nw, lanes, main = _build_sc_partials(batch, dim, num_classes)
    y32 = y.astype(jnp.int32)
    tail_tbl = centers[main:]
    partials = sc_partials(y32, hidden, centers.T, tail_tbl)
    scale = _LAMBDA_C / 2.0 / batch
    loss = pl.pallas_call(
        functools.partial(_finish_body, scale * scale),
        out_shape=jax.ShapeDtypeStruct((1, 1), jnp.float32),
    )(partials)
    return loss[0, 0]


# speedup vs baseline: 1.3027x; 1.3027x over previous
import functools

import jax
import jax.numpy as jnp
from jax import lax
from jax.experimental import pallas as pl
from jax.experimental.pallas import tpu as pltpu
from jax.experimental.pallas import tpu_sc as plsc

_LAMBDA_C = 1.0
_CH = 256


@functools.lru_cache(maxsize=None)
def _build_sc_partials(batch: int, dim: int, num_classes: int):
    info = plsc.get_sparse_core_info()
    nc, ns, lanes = info.num_cores, info.num_subcores, info.num_lanes
    nw = nc * ns
    assert dim == 64 and lanes == 16
    main = (num_classes // 128) * 128
    tail = num_classes - main
    nch = main // _CH
    nt = (nch + nw - 1) // nw

    mesh = plsc.VectorSubcoreMesh(core_axis_name="c", subcore_axis_name="s")

    @functools.partial(
        pl.kernel,
        mesh=mesh,
        out_type=jax.ShapeDtypeStruct((nw * lanes,), jnp.float32),
        compiler_params=pltpu.CompilerParams(needs_layout_passes=False),
        scratch_types=[
            pltpu.VMEM((batch,), jnp.int32),
            pltpu.VMEM((batch,), jnp.int32),
            pltpu.VMEM((batch,), jnp.int32),
            pltpu.VMEM((batch,), jnp.int32),
            pltpu.VMEM((dim, _CH), jnp.float32),
            pltpu.VMEM((dim, _CH), jnp.float32),
            pltpu.VMEM((tail, dim), jnp.float32),
            pltpu.VMEM((lanes, dim), jnp.float32),
            pltpu.VMEM((lanes, dim), jnp.float32),
            pltpu.VMEM((lanes,), jnp.float32),
            pltpu.HBM((lanes, dim), jnp.float32),
            pltpu.SemaphoreType.DMA,
            pltpu.SemaphoreType.DMA,
            pltpu.SemaphoreType.DMA,
        ],
    )
    def sc_partials(y_hbm, hidden_hbm, centers_t_hbm, tail_hbm, out_hbm,
                    hits_c, hits_b, list_c, list_b, ck0, ck1, tail_v,
                    hrow_v, cst_v, acc_v, dummy_hr, sem0, sem1, hsem):
        wid = lax.axis_index("s") * nc + lax.axis_index("c")
        y_v = hits_c
        pltpu.sync_copy(y_hbm, y_v)
        pltpu.sync_copy(tail_hbm, tail_v)
        lane_iota = lax.iota(jnp.int32, lanes)

        wid_vec = lax.broadcast(wid.astype(jnp.int32), (lanes,))
        is_last_w = wid_vec == (nw - 1)
        ones_i = jnp.ones((lanes,), jnp.int32)
        zeros_i = jnp.zeros((lanes,), jnp.int32)
        zeros_f = jnp.zeros((lanes,), jnp.float32)
        ones_f = jnp.ones((lanes,), jnp.float32)

        def bucket(i, cntv):
            yv = y_v[pl.ds(i * lanes, lanes)]
            cid = yv // _CH
            m = jnp.logical_or(
                jnp.logical_and(cid % nw == wid_vec, yv < main),
                jnp.logical_and(yv >= main, is_last_w))
            mi = jnp.where(m, ones_i, zeros_i)
            pos = cntv + plsc.cumsum(mi) - 1
            plsc.store_scatter(list_c, [pos], yv, mask=m)
            plsc.store_scatter(list_b, [pos], i * lanes + lane_iota, mask=m)
            return cntv + plsc.all_reduce_population_count(m)

        cnt_vec = lax.fori_loop(0, batch // lanes, bucket, zeros_i)
        cnt = cnt_vec[0]
        ng = (cnt + lanes - 1) // lanes

        def compact(lo, width):
            def grp(g, hv):
                c_vec = list_c[pl.ds(g * lanes, lanes)]
                b_vec = list_b[pl.ds(g * lanes, lanes)]
                m = jnp.logical_and(
                    jnp.logical_and(c_vec >= lo, c_vec < lo + width),
                    (g * lanes + lane_iota) < cnt_vec)
                mi = jnp.where(m, ones_i, zeros_i)
                pos = hv + plsc.cumsum(mi) - 1
                plsc.store_scatter(hits_c, [pos], c_vec - lo, mask=m)
                plsc.store_scatter(hits_b, [pos], b_vec, mask=m)
                return hv + plsc.all_reduce_population_count(m)

            return lax.fori_loop(0, ng, grp, zeros_i)

        def hit_group(h, acc, hcnt_vec, width, table_v, idx_of):
            col = hits_c[pl.ds(h * lanes, lanes)]
            b_vec = hits_b[pl.ds(h * lanes, lanes)]
            m = (h * lanes + lane_iota) < hcnt_vec
            wf = jnp.where(m, ones_f, zeros_f)
            b_safe = jnp.clip(b_vec, 0, batch - 1)
            col = jnp.clip(col, 0, width - 1)
            for t in range(lanes):
                pltpu.async_copy(hidden_hbm.at[b_safe[t]],
                                 hrow_v.at[t], hsem)
            for t in range(lanes):
                col_t = col[t]
                for k in range(dim // lanes):
                    cv = plsc.load_gather(
                        table_v, idx_of(col_t, k * lanes + lane_iota))
                    cst_v[t, pl.ds(k * lanes, lanes)] = cv
            pltpu.make_async_copy(dummy_hr, hrow_v, hsem).wait()
            for t in range(lanes):
                w_t = wf[t]
                for k in range(dim // lanes):
                    hv = hrow_v[t, pl.ds(k * lanes, lanes)]
                    cv = cst_v[t, pl.ds(k * lanes, lanes)]
                    d = (hv - cv) * w_t
                    acc = acc + d * d
            return acc

        def fire_chunk(slot, buf, sem):
            c_idx = wid + slot * nw

            @pl.when(c_idx < nch)
            def _():
                c0 = pl.multiple_of(c_idx * _CH, _CH)
                for i in range(dim // 8):
                    pltpu.async_copy(
                        centers_t_hbm.at[pl.ds(8 * i, 8), pl.ds(c0, _CH)],
                        buf.at[pl.ds(8 * i, 8), :], sem)

        fire_chunk(0, ck0, sem0)
        zero = jnp.zeros((lanes,), jnp.float32)

        def chunk_body(t, buf, sem, nbuf, nsem, acc):
            c_idx = wid + t * nw

            def do(acc):
                for i in range(dim // 8):
                    pltpu.make_async_copy(
                        centers_t_hbm.at[pl.ds(0, 8), pl.ds(0, _CH)],
                        buf.at[pl.ds(8 * i, 8), :], sem).wait()
                fire_chunk(t + 1, nbuf, nsem)
                hcnt_vec = compact(c_idx * _CH, _CH)
                ngh = (hcnt_vec[0] + lanes - 1) // lanes

                def grp(h, a):
                    return hit_group(
                        h, a, hcnt_vec, _CH, buf,
                        lambda col_t, dvec: [dvec, lax.broadcast(
                            col_t, (lanes,))])

                return lax.fori_loop(0, ngh, grp, acc)

            return lax.cond(c_idx < nch, do, lambda a: a, acc)

        def pair_body(t2, acc):
            acc = chunk_body(2 * t2, ck0, sem0, ck1, sem1, acc)
            acc = chunk_body(2 * t2 + 1, ck1, sem1, ck0, sem0, acc)
            return acc

        acc = lax.fori_loop(0, (nt + 1) // 2, pair_body, zero)

        tail_cnt = compact(main, tail)
        ngt = (tail_cnt[0] + lanes - 1) // lanes

        def tail_grp(h, a):
            return hit_group(
                h, a, tail_cnt, tail, tail_v,
                lambda col_t, dvec: [lax.broadcast(col_t, (lanes,)), dvec])

        acc = lax.fori_loop(0, ngt, tail_grp, acc)

        acc_v[...] = acc
        pltpu.sync_copy(acc_v, out_hbm.at[pl.ds(wid * lanes, lanes)])

    return sc_partials, nw, lanes, main


def _finish_body(scale_sq, p_ref, o_ref):
    s = jnp.sum(p_ref[...])
    o_ref[...] = jnp.broadcast_to(jnp.sqrt(s * scale_sq), (1, 1))


def kernel(y, hidden, centers):
    batch, dim = hidden.shape
    num_classes = centers.shape[0]
    sc_partials, nw, lanes, main = _build_sc_partials(batch, dim, num_classes)
    y32 = y.astype(jnp.int32)
    tail_tbl = centers[main:]
    partials = sc_partials(y32, hidden, centers.T, tail_tbl)
    scale = _LAMBDA_C / 2.0 / batch
    loss = pl.pallas_call(
        functools.partial(_finish_body, scale * scale),
        out_shape=jax.ShapeDtypeStruct((1, 1), jnp.float32),
    )(partials)
    return loss[0, 0]


# speedup vs baseline: 3.6716x; 2.8186x over previous
import functools

import jax
import jax.numpy as jnp
from jax import lax
from jax.experimental import pallas as pl
from jax.experimental.pallas import tpu as pltpu
from jax.experimental.pallas import tpu_sc as plsc

_LAMBDA_C = 1.0
_CH = 256
_NSB = 8


@functools.lru_cache(maxsize=None)
def _build_sc_partials(batch: int, dim: int, num_classes: int):
    info = plsc.get_sparse_core_info()
    nc, ns, lanes = info.num_cores, info.num_subcores, info.num_lanes
    nw = nc * ns
    assert dim == 64 and lanes == 16
    main = (num_classes // 128) * 128
    tail = num_classes - main
    nch = main // _CH
    nt = (nch + nw - 1) // nw
    sb_span = (nt + _NSB - 1) // _NSB
    cap = batch + _NSB * lanes

    mesh = plsc.VectorSubcoreMesh(core_axis_name="c", subcore_axis_name="s")

    @functools.partial(
        pl.kernel,
        mesh=mesh,
        out_type=jax.ShapeDtypeStruct((nw * lanes,), jnp.float32),
        compiler_params=pltpu.CompilerParams(needs_layout_passes=False),
        scratch_types=[
            pltpu.VMEM((cap,), jnp.int32),
            pltpu.VMEM((cap,), jnp.int32),
            pltpu.VMEM((cap,), jnp.int32),
            pltpu.VMEM((cap,), jnp.int32),
            pltpu.VMEM((lanes,), jnp.int32),
            pltpu.VMEM((lanes,), jnp.int32),
            pltpu.VMEM((dim, _CH), jnp.float32),
            pltpu.VMEM((dim, _CH), jnp.float32),
            pltpu.VMEM((tail, dim), jnp.float32),
            pltpu.VMEM((lanes, dim), jnp.float32),
            pltpu.VMEM((lanes, dim), jnp.float32),
            pltpu.VMEM((lanes,), jnp.float32),
            pltpu.HBM((lanes, dim), jnp.float32),
            pltpu.SemaphoreType.DMA,
            pltpu.SemaphoreType.DMA,
            pltpu.SemaphoreType.DMA,
        ],
    )
    def sc_partials(y_hbm, hidden_hbm, centers_t_hbm, tail_hbm, out_hbm,
                    sub_c, sub_b, list_c, list_b, sbst_v, sbcn_v, ck0, ck1,
                    tail_v, hrow_v, cst_v, acc_v, dummy_hr,
                    sem0, sem1, hsem):
        wid = lax.axis_index("s") * nc + lax.axis_index("c")
        y_v = sub_c
        pltpu.sync_copy(y_hbm, y_v.at[pl.ds(0, batch)])
        pltpu.sync_copy(tail_hbm, tail_v)
        lane_iota = lax.iota(jnp.int32, lanes)

        wid_vec = lax.broadcast(wid.astype(jnp.int32), (lanes,))
        is_last_w = wid_vec == (nw - 1)
        ones_i = jnp.ones((lanes,), jnp.int32)
        zeros_i = jnp.zeros((lanes,), jnp.int32)
        zeros_f = jnp.zeros((lanes,), jnp.float32)
        ones_f = jnp.ones((lanes,), jnp.float32)

        def bucket(i, cntv):
            yv = y_v[pl.ds(i * lanes, lanes)]
            cid = yv // _CH
            m = jnp.logical_or(
                jnp.logical_and(cid % nw == wid_vec, yv < main),
                jnp.logical_and(yv >= main, is_last_w))
            mi = jnp.where(m, ones_i, zeros_i)
            pos = cntv + plsc.cumsum(mi) - 1
            plsc.store_scatter(list_c, [pos], yv, mask=m)
            plsc.store_scatter(list_b, [pos], i * lanes + lane_iota, mask=m)
            return cntv + plsc.all_reduce_population_count(m)

        cnt_vec = lax.fori_loop(0, batch // lanes, bucket, zeros_i)
        ng = (cnt_vec[0] + lanes - 1) // lanes

        off_vec = zeros_i
        for sb in range(_NSB):
            plsc.store_scatter(sbst_v, [lane_iota], off_vec,
                               mask=lane_iota == sb)

            def rep(g, hv, sb=sb, off_vec=off_vec):
                c_vec = list_c[pl.ds(g * lanes, lanes)]
                b_vec = list_b[pl.ds(g * lanes, lanes)]
                slot = (c_vec // _CH) // nw
                m = jnp.logical_and(
                    slot // sb_span == sb,
                    (g * lanes + lane_iota) < cnt_vec)
                mi = jnp.where(m, ones_i, zeros_i)
                pos = off_vec + hv + plsc.cumsum(mi) - 1
                plsc.store_scatter(sub_c, [pos], c_vec, mask=m)
                plsc.store_scatter(sub_b, [pos], b_vec, mask=m)
                return hv + plsc.all_reduce_population_count(m)

            sbcnt = lax.fori_loop(0, ng, rep, zeros_i)
            plsc.store_scatter(sbcn_v, [lane_iota], sbcnt,
                               mask=lane_iota == sb)
            off_vec = off_vec + ((sbcnt + lanes - 1) // lanes) * lanes

        hits_c, hits_b = list_c, list_b

        def compact(lo, width, start_s, scnt_vec):
            ngs = (scnt_vec[0] + lanes - 1) // lanes

            def grp(g, hv):
                base = start_s + g * lanes
                c_vec = sub_c[pl.ds(base, lanes)]
                b_vec = sub_b[pl.ds(base, lanes)]
                m = jnp.logical_and(
                    jnp.logical_and(c_vec >= lo, c_vec < lo + width),
                    (g * lanes + lane_iota) < scnt_vec)
                mi = jnp.where(m, ones_i, zeros_i)
                pos = hv + plsc.cumsum(mi) - 1
                plsc.store_scatter(hits_c, [pos], c_vec - lo, mask=m)
                plsc.store_scatter(hits_b, [pos], b_vec, mask=m)
                return hv + plsc.all_reduce_population_count(m)

            return lax.fori_loop(0, ngs, grp, zeros_i)

        def hit_group(h, acc, hcnt_vec, width, table_v, idx_of):
            col = hits_c[pl.ds(h * lanes, lanes)]
            b_vec = hits_b[pl.ds(h * lanes, lanes)]
            m = (h * lanes + lane_iota) < hcnt_vec
            wf = jnp.where(m, ones_f, zeros_f)
            b_safe = jnp.clip(b_vec, 0, batch - 1)
            col = jnp.clip(col, 0, width - 1)
            for t in range(lanes):
                pltpu.async_copy(hidden_hbm.at[b_safe[t]],
                                 hrow_v.at[t], hsem)
            for t in range(lanes):
                col_t = col[t]
                for k in range(dim // lanes):
                    cv = plsc.load_gather(
                        table_v, idx_of(col_t, k * lanes + lane_iota))
                    cst_v[t, pl.ds(k * lanes, lanes)] = cv
            pltpu.make_async_copy(dummy_hr, hrow_v, hsem).wait()
            for t in range(lanes):
                w_t = wf[t]
                for k in range(dim // lanes):
                    hv = hrow_v[t, pl.ds(k * lanes, lanes)]
                    cv = cst_v[t, pl.ds(k * lanes, lanes)]
                    d = (hv - cv) * w_t
                    acc = acc + d * d
            return acc

        def fire_chunk(slot, buf, sem):
            c_idx = wid + slot * nw

            @pl.when(c_idx < nch)
            def _():
                c0 = pl.multiple_of(c_idx * _CH, _CH)
                for i in range(dim // 8):
                    pltpu.async_copy(
                        centers_t_hbm.at[pl.ds(8 * i, 8), pl.ds(c0, _CH)],
                        buf.at[pl.ds(8 * i, 8), :], sem)

        fire_chunk(0, ck0, sem0)
        zero = jnp.zeros((lanes,), jnp.float32)

        def chunk_body(t, buf, sem, nbuf, nsem, acc):
            c_idx = wid + t * nw

            def do(acc):
                for i in range(dim // 8):
                    pltpu.make_async_copy(
                        centers_t_hbm.at[pl.ds(0, 8), pl.ds(0, _CH)],
                        buf.at[pl.ds(8 * i, 8), :], sem).wait()
                fire_chunk(t + 1, nbuf, nsem)
                sb_vec = lax.broadcast(t // sb_span, (lanes,))
                start_vec = plsc.load_gather(sbst_v, [sb_vec])
                scnt_vec = plsc.load_gather(sbcn_v, [sb_vec])
                hcnt_vec = compact(c_idx * _CH, _CH, start_vec[0], scnt_vec)
                ngh = (hcnt_vec[0] + lanes - 1) // lanes

                def grp(h, a):
                    return hit_group(
                        h, a, hcnt_vec, _CH, buf,
                        lambda col_t, dvec: [dvec, lax.broadcast(
                            col_t, (lanes,))])

                return lax.fori_loop(0, ngh, grp, acc)

            return lax.cond(c_idx < nch, do, lambda a: a, acc)

        def pair_body(t2, acc):
            acc = chunk_body(2 * t2, ck0, sem0, ck1, sem1, acc)
            acc = chunk_body(2 * t2 + 1, ck1, sem1, ck0, sem0, acc)
            return acc

        acc = lax.fori_loop(0, (nt + 1) // 2, pair_body, zero)

        sb7_vec = lax.broadcast(jnp.int32(_NSB - 1), (lanes,))
        t_start = plsc.load_gather(sbst_v, [sb7_vec])
        t_scnt = plsc.load_gather(sbcn_v, [sb7_vec])
        tail_cnt = compact(main, tail, t_start[0], t_scnt)
        ngt = (tail_cnt[0] + lanes - 1) // lanes

        def tail_grp(h, a):
            return hit_group(
                h, a, tail_cnt, tail, tail_v,
                lambda col_t, dvec: [lax.broadcast(col_t, (lanes,)), dvec])

        acc = lax.fori_loop(0, ngt, tail_grp, acc)

        acc_v[...] = acc
        pltpu.sync_copy(acc_v, out_hbm.at[pl.ds(wid * lanes, lanes)])

    return sc_partials, nw, lanes, main


def _finish_body(scale_sq, p_ref, o_ref):
    s = jnp.sum(p_ref[...])
    o_ref[...] = jnp.broadcast_to(jnp.sqrt(s * scale_sq), (1, 1))


def kernel(y, hidden, centers):
    batch, dim = hidden.shape
    num_classes = centers.shape[0]
    sc_partials, nw, lanes, main = _build_sc_partials(batch, dim, num_classes)
    y32 = y.astype(jnp.int32)
    tail_tbl = centers[main:]
    partials = sc_partials(y32, hidden, centers.T, tail_tbl)
    scale = _LAMBDA_C / 2.0 / batch
    loss = pl.pallas_call(
        functools.partial(_finish_body, scale * scale),
        out_shape=jax.ShapeDtypeStruct((1, 1), jnp.float32),
    )(partials)
    return loss[0, 0]


# speedup vs baseline: 3.9995x; 1.0893x over previous
import functools

import jax
import jax.numpy as jnp
from jax import lax
from jax.experimental import pallas as pl
from jax.experimental.pallas import tpu as pltpu
from jax.experimental.pallas import tpu_sc as plsc

_LAMBDA_C = 1.0
_CH = 256
_NSB = 8


@functools.lru_cache(maxsize=None)
def _build_sc_partials(batch: int, dim: int, num_classes: int):
    info = plsc.get_sparse_core_info()
    nc, ns, lanes = info.num_cores, info.num_subcores, info.num_lanes
    nw = nc * ns
    assert dim == 64 and lanes == 16
    main = (num_classes // 128) * 128
    tail = num_classes - main
    nch = main // _CH
    nt = (nch + nw - 1) // nw
    sb_span = (nt + _NSB - 1) // _NSB
    cap = batch + _NSB * lanes

    mesh = plsc.VectorSubcoreMesh(core_axis_name="c", subcore_axis_name="s")

    @functools.partial(
        pl.kernel,
        mesh=mesh,
        out_type=jax.ShapeDtypeStruct((nw * lanes,), jnp.float32),
        compiler_params=pltpu.CompilerParams(needs_layout_passes=False),
        scratch_types=[
            pltpu.VMEM((cap,), jnp.int32),
            pltpu.VMEM((cap,), jnp.int32),
            pltpu.VMEM((cap,), jnp.int32),
            pltpu.VMEM((cap,), jnp.int32),
            pltpu.VMEM((lanes,), jnp.int32),
            pltpu.VMEM((lanes,), jnp.int32),
            pltpu.VMEM((dim, _CH), jnp.float32),
            pltpu.VMEM((dim, _CH), jnp.float32),
            pltpu.VMEM((tail, dim), jnp.float32),
            pltpu.VMEM((lanes, dim), jnp.float32),
            pltpu.VMEM((lanes, dim), jnp.float32),
            pltpu.VMEM((lanes,), jnp.float32),
            pltpu.HBM((lanes, dim), jnp.float32),
            pltpu.SemaphoreType.DMA,
            pltpu.SemaphoreType.DMA,
            pltpu.SemaphoreType.DMA,
        ],
    )
    def sc_partials(y_hbm, hidden_hbm, centers_t_hbm, tail_hbm, out_hbm,
                    sub_c, sub_b, list_c, list_b, sbst_v, sbcn_v, ck0, ck1,
                    tail_v, hrow_v, cst_v, acc_v, dummy_hr,
                    sem0, sem1, hsem):
        wid = lax.axis_index("s") * nc + lax.axis_index("c")
        y_v = sub_c
        pltpu.sync_copy(y_hbm, y_v.at[pl.ds(0, batch)])
        pltpu.sync_copy(tail_hbm, tail_v)
        lane_iota = lax.iota(jnp.int32, lanes)

        def _prime(slot, buf, sem):
            c_idx = wid + slot * nw

            @pl.when(c_idx < nch)
            def _():
                c0 = pl.multiple_of(c_idx * _CH, _CH)
                pltpu.async_copy(
                    centers_t_hbm.at[:, pl.ds(c0, _CH)], buf, sem)

        _prime(0, ck0, sem0)
        _prime(1, ck1, sem1)

        wid_vec = lax.broadcast(wid.astype(jnp.int32), (lanes,))
        is_last_w = wid_vec == (nw - 1)
        ones_i = jnp.ones((lanes,), jnp.int32)
        zeros_i = jnp.zeros((lanes,), jnp.int32)
        zeros_f = jnp.zeros((lanes,), jnp.float32)
        ones_f = jnp.ones((lanes,), jnp.float32)

        def bucket(i, cntv):
            yv = y_v[pl.ds(i * lanes, lanes)]
            cid = yv // _CH
            m = jnp.logical_or(
                jnp.logical_and(cid % nw == wid_vec, yv < main),
                jnp.logical_and(yv >= main, is_last_w))
            mi = jnp.where(m, ones_i, zeros_i)
            pos = cntv + plsc.cumsum(mi) - 1
            plsc.store_scatter(list_c, [pos], yv, mask=m)
            plsc.store_scatter(list_b, [pos], i * lanes + lane_iota, mask=m)
            return cntv + plsc.all_reduce_population_count(m)

        cnt_vec = lax.fori_loop(0, batch // lanes, bucket, zeros_i)
        ng = (cnt_vec[0] + lanes - 1) // lanes

        off_vec = zeros_i
        for sb in range(_NSB):
            plsc.store_scatter(sbst_v, [lane_iota], off_vec,
                               mask=lane_iota == sb)

            def rep(g, hv, sb=sb, off_vec=off_vec):
                c_vec = list_c[pl.ds(g * lanes, lanes)]
                b_vec = list_b[pl.ds(g * lanes, lanes)]
                slot = (c_vec // _CH) // nw
                m = jnp.logical_and(
                    slot // sb_span == sb,
                    (g * lanes + lane_iota) < cnt_vec)
                mi = jnp.where(m, ones_i, zeros_i)
                pos = off_vec + hv + plsc.cumsum(mi) - 1
                plsc.store_scatter(sub_c, [pos], c_vec, mask=m)
                plsc.store_scatter(sub_b, [pos], b_vec, mask=m)
                return hv + plsc.all_reduce_population_count(m)

            sbcnt = lax.fori_loop(0, ng, rep, zeros_i)
            plsc.store_scatter(sbcn_v, [lane_iota], sbcnt,
                               mask=lane_iota == sb)
            off_vec = off_vec + ((sbcnt + lanes - 1) // lanes) * lanes

        hits_c, hits_b = list_c, list_b

        def compact(lo, width, start_s, scnt_vec):
            ngs = (scnt_vec[0] + lanes - 1) // lanes

            def grp(g, hv):
                base = start_s + g * lanes
                c_vec = sub_c[pl.ds(base, lanes)]
                b_vec = sub_b[pl.ds(base, lanes)]
                m = jnp.logical_and(
                    jnp.logical_and(c_vec >= lo, c_vec < lo + width),
                    (g * lanes + lane_iota) < scnt_vec)
                mi = jnp.where(m, ones_i, zeros_i)
                pos = hv + plsc.cumsum(mi) - 1
                plsc.store_scatter(hits_c, [pos], c_vec - lo, mask=m)
                plsc.store_scatter(hits_b, [pos], b_vec, mask=m)
                return hv + plsc.all_reduce_population_count(m)

            return lax.fori_loop(0, ngs, grp, zeros_i)

        def hit_group(h, acc, hcnt_vec, width, table_v, idx_of):
            col = hits_c[pl.ds(h * lanes, lanes)]
            b_vec = hits_b[pl.ds(h * lanes, lanes)]
            m = (h * lanes + lane_iota) < hcnt_vec
            wf = jnp.where(m, ones_f, zeros_f)
            b_safe = jnp.clip(b_vec, 0, batch - 1)
            col = jnp.clip(col, 0, width - 1)
            for t in range(lanes):
                pltpu.async_copy(hidden_hbm.at[b_safe[t]],
                                 hrow_v.at[t], hsem)
            for t in range(lanes):
                col_t = col[t]
                for k in range(dim // lanes):
                    cv = plsc.load_gather(
                        table_v, idx_of(col_t, k * lanes + lane_iota))
                    cst_v[t, pl.ds(k * lanes, lanes)] = cv
            pltpu.make_async_copy(dummy_hr, hrow_v, hsem).wait()
            for t in range(lanes):
                w_t = wf[t]
                for k in range(dim // lanes):
                    hv = hrow_v[t, pl.ds(k * lanes, lanes)]
                    cv = cst_v[t, pl.ds(k * lanes, lanes)]
                    d = (hv - cv) * w_t
                    acc = acc + d * d
            return acc

        def fire_chunk(slot, buf, sem):
            c_idx = wid + slot * nw

            @pl.when(c_idx < nch)
            def _():
                c0 = pl.multiple_of(c_idx * _CH, _CH)
                pltpu.async_copy(
                    centers_t_hbm.at[:, pl.ds(c0, _CH)], buf, sem)

        zero = jnp.zeros((lanes,), jnp.float32)

        def chunk_body(t, buf, sem, nbuf, nsem, acc):
            c_idx = wid + t * nw

            def do(acc):
                pltpu.make_async_copy(
                    centers_t_hbm.at[:, pl.ds(0, _CH)], buf, sem).wait()
                sb_vec = lax.broadcast(t // sb_span, (lanes,))
                start_vec = plsc.load_gather(sbst_v, [sb_vec])
                scnt_vec = plsc.load_gather(sbcn_v, [sb_vec])
                hcnt_vec = compact(c_idx * _CH, _CH, start_vec[0], scnt_vec)
                ngh = (hcnt_vec[0] + lanes - 1) // lanes

                def grp(h, a):
                    return hit_group(
                        h, a, hcnt_vec, _CH, buf,
                        lambda col_t, dvec: [dvec, lax.broadcast(
                            col_t, (lanes,))])

                acc = lax.fori_loop(0, ngh, grp, acc)
                fire_chunk(t + 2, buf, sem)
                return acc

            return lax.cond(c_idx < nch, do, lambda a: a, acc)

        def pair_body(t2, acc):
            acc = chunk_body(2 * t2, ck0, sem0, ck1, sem1, acc)
            acc = chunk_body(2 * t2 + 1, ck1, sem1, ck0, sem0, acc)
            return acc

        acc = lax.fori_loop(0, (nt + 1) // 2, pair_body, zero)

        sb7_vec = lax.broadcast(jnp.int32(_NSB - 1), (lanes,))
        t_start = plsc.load_gather(sbst_v, [sb7_vec])
        t_scnt = plsc.load_gather(sbcn_v, [sb7_vec])
        tail_cnt = compact(main, tail, t_start[0], t_scnt)
        ngt = (tail_cnt[0] + lanes - 1) // lanes

        def tail_grp(h, a):
            return hit_group(
                h, a, tail_cnt, tail, tail_v,
                lambda col_t, dvec: [lax.broadcast(col_t, (lanes,)), dvec])

        acc = lax.fori_loop(0, ngt, tail_grp, acc)

        acc_v[...] = acc
        pltpu.sync_copy(acc_v, out_hbm.at[pl.ds(wid * lanes, lanes)])

    return sc_partials, nw, lanes, main


def _finish_body(scale_sq, p_ref, o_ref):
    s = jnp.sum(p_ref[...])
    o_ref[...] = jnp.broadcast_to(jnp.sqrt(s * scale_sq), (1, 1))


def kernel(y, hidden, centers):
    batch, dim = hidden.shape
    num_classes = centers.shape[0]
    sc_partials, nw, lanes, main = _build_sc_partials(batch, dim, num_classes)
    y32 = y.astype(jnp.int32)
    tail_tbl = centers[main:]
    partials = sc_partials(y32, hidden, centers.T, tail_tbl)
    scale = _LAMBDA_C / 2.0 / batch
    loss = pl.pallas_call(
        functools.partial(_finish_body, scale * scale),
        out_shape=jax.ShapeDtypeStruct((1, 1), jnp.float32),
    )(partials)
    return loss[0, 0]


# speedup vs baseline: 4.0686x; 1.0173x over previous
import functools

import jax
import jax.numpy as jnp
from jax import lax
from jax.experimental import pallas as pl
from jax.experimental.pallas import tpu as pltpu
from jax.experimental.pallas import tpu_sc as plsc

_LAMBDA_C = 1.0
_CH = 256
_NSB = 8


@functools.lru_cache(maxsize=None)
def _build_sc_partials(batch: int, dim: int, num_classes: int):
    info = plsc.get_sparse_core_info()
    nc, ns, lanes = info.num_cores, info.num_subcores, info.num_lanes
    nw = nc * ns
    assert dim == 64 and lanes == 16
    main = (num_classes // 128) * 128
    tail = num_classes - main
    nch = main // _CH
    nt = (nch + nw - 1) // nw
    sb_span = (nt + _NSB - 1) // _NSB
    cap = batch + _NSB * lanes

    mesh = plsc.VectorSubcoreMesh(core_axis_name="c", subcore_axis_name="s")

    @functools.partial(
        pl.kernel,
        mesh=mesh,
        out_type=jax.ShapeDtypeStruct((nw * lanes,), jnp.float32),
        compiler_params=pltpu.CompilerParams(needs_layout_passes=False),
        scratch_types=[
            pltpu.VMEM((cap,), jnp.int32),
            pltpu.VMEM((cap,), jnp.int32),
            pltpu.VMEM((cap,), jnp.int32),
            pltpu.VMEM((cap,), jnp.int32),
            pltpu.VMEM((lanes,), jnp.int32),
            pltpu.VMEM((lanes,), jnp.int32),
            pltpu.VMEM((dim, _CH), jnp.float32),
            pltpu.VMEM((dim, _CH), jnp.float32),
            pltpu.VMEM((tail, dim), jnp.float32),
            pltpu.VMEM((lanes, dim), jnp.float32),
            pltpu.VMEM((lanes, dim), jnp.float32),
            pltpu.VMEM((lanes,), jnp.float32),
            pltpu.HBM((lanes, dim), jnp.float32),
            pltpu.SemaphoreType.DMA,
            pltpu.SemaphoreType.DMA,
            pltpu.SemaphoreType.DMA,
        ],
    )
    def sc_partials(y_hbm, hidden_hbm, centers_t_hbm, tail_hbm, out_hbm,
                    sub_c, sub_b, list_c, list_b, sbst_v, sbcn_v, ck0, ck1,
                    tail_v, hrow_v, cst_v, acc_v, dummy_hr,
                    sem0, sem1, hsem):
        wid = lax.axis_index("s") * nc + lax.axis_index("c")
        y_v = sub_c
        pltpu.sync_copy(y_hbm, y_v.at[pl.ds(0, batch)])
        pltpu.sync_copy(tail_hbm, tail_v)
        lane_iota = lax.iota(jnp.int32, lanes)

        def _prime(slot, buf, sem):
            c_idx = wid + slot * nw

            @pl.when(c_idx < nch)
            def _():
                c0 = pl.multiple_of(c_idx * _CH, _CH)
                pltpu.async_copy(
                    centers_t_hbm.at[:, pl.ds(c0, _CH)], buf, sem)

        _prime(0, ck0, sem0)
        _prime(1, ck1, sem1)

        wid_vec = lax.broadcast(wid.astype(jnp.int32), (lanes,))
        is_last_w = wid_vec == (nw - 1)
        ones_i = jnp.ones((lanes,), jnp.int32)
        zeros_i = jnp.zeros((lanes,), jnp.int32)
        zeros_f = jnp.zeros((lanes,), jnp.float32)
        ones_f = jnp.ones((lanes,), jnp.float32)

        assert _CH & (_CH - 1) == 0 and nw & (nw - 1) == 0
        assert sb_span & (sb_span - 1) == 0
        ch_sh = _CH.bit_length() - 1
        nw_sh = nw.bit_length() - 1
        sb_sh = sb_span.bit_length() - 1

        def bucket(i, cntv):
            yv = y_v[pl.ds(i * lanes, lanes)]
            cid = yv >> ch_sh
            m = jnp.logical_or(
                jnp.logical_and((cid & (nw - 1)) == wid_vec, yv < main),
                jnp.logical_and(yv >= main, is_last_w))
            mi = jnp.where(m, ones_i, zeros_i)
            pos = cntv + plsc.cumsum(mi) - 1
            plsc.store_scatter(list_c, [pos], yv, mask=m)
            plsc.store_scatter(list_b, [pos], i * lanes + lane_iota, mask=m)
            return cntv + plsc.all_reduce_population_count(m)

        cnt_vec = lax.fori_loop(0, batch // lanes, bucket, zeros_i,
                                unroll=2)
        ng = (cnt_vec[0] + lanes - 1) // lanes

        off_vec = zeros_i
        for sb in range(_NSB):
            plsc.store_scatter(sbst_v, [lane_iota], off_vec,
                               mask=lane_iota == sb)

            def rep(g, hv, sb=sb, off_vec=off_vec):
                c_vec = list_c[pl.ds(g * lanes, lanes)]
                b_vec = list_b[pl.ds(g * lanes, lanes)]
                m = jnp.logical_and(
                    (c_vec >> (ch_sh + nw_sh + sb_sh)) == sb,
                    (g * lanes + lane_iota) < cnt_vec)
                mi = jnp.where(m, ones_i, zeros_i)
                pos = off_vec + hv + plsc.cumsum(mi) - 1
                plsc.store_scatter(sub_c, [pos], c_vec, mask=m)
                plsc.store_scatter(sub_b, [pos], b_vec, mask=m)
                return hv + plsc.all_reduce_population_count(m)

            sbcnt = lax.fori_loop(0, ng, rep, zeros_i)
            plsc.store_scatter(sbcn_v, [lane_iota], sbcnt,
                               mask=lane_iota == sb)
            off_vec = off_vec + ((sbcnt + lanes - 1) // lanes) * lanes

        hits_c, hits_b = list_c, list_b

        def compact(lo, width, start_s, scnt_vec):
            ngs = (scnt_vec[0] + lanes - 1) // lanes

            def grp(g, hv):
                base = start_s + g * lanes
                c_vec = sub_c[pl.ds(base, lanes)]
                b_vec = sub_b[pl.ds(base, lanes)]
                m = jnp.logical_and(
                    jnp.logical_and(c_vec >= lo, c_vec < lo + width),
                    (g * lanes + lane_iota) < scnt_vec)
                mi = jnp.where(m, ones_i, zeros_i)
                pos = hv + plsc.cumsum(mi) - 1
                plsc.store_scatter(hits_c, [pos], c_vec - lo, mask=m)
                plsc.store_scatter(hits_b, [pos], b_vec, mask=m)
                return hv + plsc.all_reduce_population_count(m)

            return lax.fori_loop(0, ngs, grp, zeros_i)

        def fire_rows(h):
            b_vec = hits_b[pl.ds(h * lanes, lanes)]
            b_safe = jnp.clip(b_vec, 0, batch - 1)
            for t in range(lanes):
                pltpu.async_copy(hidden_hbm.at[b_safe[t]],
                                 hrow_v.at[t], hsem)

        def hit_group(h, acc, hcnt_vec, ngh, width, table_v, idx_of):
            col = hits_c[pl.ds(h * lanes, lanes)]
            m = (h * lanes + lane_iota) < hcnt_vec
            wf = jnp.where(m, ones_f, zeros_f)
            col = jnp.clip(col, 0, width - 1)
            for t in range(lanes):
                col_t = col[t]
                for k in range(dim // lanes):
                    cv = plsc.load_gather(
                        table_v, idx_of(col_t, k * lanes + lane_iota))
                    cst_v[t, pl.ds(k * lanes, lanes)] = cv
            pltpu.make_async_copy(dummy_hr, hrow_v, hsem).wait()
            for t in range(lanes):
                w_t = wf[t]
                for k in range(dim // lanes):
                    hv = hrow_v[t, pl.ds(k * lanes, lanes)]
                    cv = cst_v[t, pl.ds(k * lanes, lanes)]
                    d = (hv - cv) * w_t
                    acc = acc + d * d

            @pl.when(h + 1 < ngh)
            def _():
                fire_rows(h + 1)

            return acc

        def fire_chunk(slot, buf, sem):
            c_idx = wid + slot * nw

            @pl.when(c_idx < nch)
            def _():
                c0 = pl.multiple_of(c_idx * _CH, _CH)
                pltpu.async_copy(
                    centers_t_hbm.at[:, pl.ds(c0, _CH)], buf, sem)

        zero = jnp.zeros((lanes,), jnp.float32)

        def chunk_body(t, buf, sem, nbuf, nsem, acc):
            c_idx = wid + t * nw

            def do(acc):
                sb_vec = lax.broadcast(t // sb_span, (lanes,))
                start_vec = plsc.load_gather(sbst_v, [sb_vec])
                scnt_vec = plsc.load_gather(sbcn_v, [sb_vec])
                hcnt_vec = compact(c_idx * _CH, _CH, start_vec[0], scnt_vec)
                ngh = (hcnt_vec[0] + lanes - 1) // lanes

                @pl.when(ngh > 0)
                def _():
                    fire_rows(0)

                pltpu.make_async_copy(
                    centers_t_hbm.at[:, pl.ds(0, _CH)], buf, sem).wait()

                def grp(h, a):
                    return hit_group(
                        h, a, hcnt_vec, ngh, _CH, buf,
                        lambda col_t, dvec: [dvec, lax.broadcast(
                            col_t, (lanes,))])

                acc = lax.fori_loop(0, ngh, grp, acc)
                fire_chunk(t + 2, buf, sem)
                return acc

            return lax.cond(c_idx < nch, do, lambda a: a, acc)

        def pair_body(t2, acc):
            acc = chunk_body(2 * t2, ck0, sem0, ck1, sem1, acc)
            acc = chunk_body(2 * t2 + 1, ck1, sem1, ck0, sem0, acc)
            return acc

        acc = lax.fori_loop(0, (nt + 1) // 2, pair_body, zero)

        sb7_vec = lax.broadcast(jnp.int32(_NSB - 1), (lanes,))
        t_start = plsc.load_gather(sbst_v, [sb7_vec])
        t_scnt = plsc.load_gather(sbcn_v, [sb7_vec])
        tail_cnt = compact(main, tail, t_start[0], t_scnt)
        ngt = (tail_cnt[0] + lanes - 1) // lanes

        @pl.when(ngt > 0)
        def _():
            fire_rows(0)

        def tail_grp(h, a):
            return hit_group(
                h, a, tail_cnt, ngt, tail, tail_v,
                lambda col_t, dvec: [lax.broadcast(col_t, (lanes,)), dvec])

        acc = lax.fori_loop(0, ngt, tail_grp, acc)

        acc_v[...] = acc
        pltpu.sync_copy(acc_v, out_hbm.at[pl.ds(wid * lanes, lanes)])

    return sc_partials, nw, lanes, main


def _finish_body(scale_sq, p_ref, o_ref):
    s = jnp.sum(p_ref[...])
    o_ref[...] = jnp.broadcast_to(jnp.sqrt(s * scale_sq), (1, 1))


def kernel(y, hidden, centers):
    batch, dim = hidden.shape
    num_classes = centers.shape[0]
    sc_partials, nw, lanes, main = _build_sc_partials(batch, dim, num_classes)
    y32 = y.astype(jnp.int32)
    tail_tbl = centers[main:]
    partials = sc_partials(y32, hidden, centers.T, tail_tbl)
    scale = _LAMBDA_C / 2.0 / batch
    loss = pl.pallas_call(
        functools.partial(_finish_body, scale * scale),
        out_shape=jax.ShapeDtypeStruct((1, 1), jnp.float32),
    )(partials)
    return loss[0, 0]
